# Initial kernel scaffold; baseline (speedup 1.0000x reference)
import jax
import jax.numpy as jnp
from jax.experimental import pallas as pl


def kernel(x, edge_index, W_proj, Wq, bq, Wk, bk, Wv, bv, gamma, beta):
    raise NotImplementedError("write your pallas kernel here")


# speedup vs baseline: 10.0759x; 10.0759x over previous
import functools
import math

import jax
import jax.numpy as jnp
from jax import lax
from jax.experimental import pallas as pl
from jax.experimental.pallas import tpu as pltpu
from jax.experimental.pallas import tpu_sc as plsc

N = 10000
E = 320000
D = 128
W = 144
NPAD = 10240
NSUB = 16
EDGES_PER_SUB = E // NSUB
CHUNK = 80
NCHUNK = EDGES_PER_SUB // CHUNK
ROWS_PER_SUB = NPAD // NSUB
INV_SQRT_HD = 1.0 / math.sqrt(16.0)


def _tc_pre_body(x_ref, wp_ref, wq_ref, bq_ref, wk_ref, bk_ref, wv_ref,
                 bv_ref, ta_ref, tb_ref):
    xv = x_ref[...]
    h = jnp.dot(xv, wp_ref[...], preferred_element_type=jnp.float32)
    q = jnp.dot(h, wq_ref[...], preferred_element_type=jnp.float32) + bq_ref[...]
    k = jnp.dot(h, wk_ref[...], preferred_element_type=jnp.float32) + bk_ref[...]
    v = jnp.dot(h, wv_ref[...], preferred_element_type=jnp.float32) + bv_ref[...]
    a = jnp.sum(q * k, axis=1, keepdims=True) * INV_SQRT_HD
    wa = jnp.exp(a - jnp.max(a))
    ones = jnp.ones((N, 1), jnp.float32)
    zpad14 = jnp.zeros((N, 14), jnp.float32)
    zpad16 = jnp.zeros((N, 16), jnp.float32)
    ta_ref[...] = jnp.concatenate([v * wa, wa, ones, zpad14], axis=1)
    tb_ref[...] = jnp.concatenate([h, zpad16], axis=1)


_tc_pre = pl.pallas_call(
    _tc_pre_body,
    out_shape=[
        jax.ShapeDtypeStruct((N, W), jnp.float32),
        jax.ShapeDtypeStruct((N, W), jnp.float32),
    ],
)


def _sc_edge_body(ta_hbm, tb_hbm, src_hbm, dst_hbm, outa_hbm, outb_hbm,
                  acc, src_v, dst_v, rows_v, zbuf, sem):
    cid = lax.axis_index("c")
    sid = lax.axis_index("s")

    for r in range(16):
        for c in range(W // 16):
            zbuf[r, pl.ds(c * 16, 16)] = jnp.zeros((16,), jnp.float32)
    for j in range(ROWS_PER_SUB // 16):
        pltpu.sync_copy(zbuf, acc.at[pl.ds(sid * ROWS_PER_SUB + j * 16, 16)])
    plsc.subcore_barrier()

    def body(i, carry):
        base = sid * EDGES_PER_SUB + i * CHUNK
        pltpu.sync_copy(src_hbm.at[pl.ds(base, CHUNK)], src_v)
        pltpu.sync_copy(dst_hbm.at[pl.ds(base, CHUNK)], dst_v)

        @pl.when(cid == 0)
        def _():
            pltpu.async_copy(ta_hbm.at[src_v], rows_v, sem).wait()

        @pl.when(cid == 1)
        def _():
            pltpu.async_copy(tb_hbm.at[src_v], rows_v, sem).wait()

        pltpu.sync_copy(rows_v, acc.at[dst_v], add=True)
        return carry

    lax.fori_loop(0, NCHUNK, body, 0)
    plsc.subcore_barrier()

    dst_slice = pl.ds(sid * ROWS_PER_SUB, ROWS_PER_SUB)

    @pl.when(cid == 0)
    def _():
        pltpu.sync_copy(acc.at[dst_slice], outa_hbm.at[dst_slice])

    @pl.when(cid == 1)
    def _():
        pltpu.sync_copy(acc.at[dst_slice], outb_hbm.at[dst_slice])


_sc_edge = functools.partial(
    pl.kernel,
    out_type=[
        jax.ShapeDtypeStruct((NPAD, W), jnp.float32),
        jax.ShapeDtypeStruct((NPAD, W), jnp.float32),
    ],
    mesh=plsc.VectorSubcoreMesh(core_axis_name="c", subcore_axis_name="s"),
    scratch_types=[
        pltpu.VMEM_SHARED((NPAD, W), jnp.float32),
        pltpu.VMEM((CHUNK,), jnp.int32),
        pltpu.VMEM((CHUNK,), jnp.int32),
        pltpu.VMEM((CHUNK, W), jnp.float32),
        pltpu.VMEM((16, W), jnp.float32),
        pltpu.SemaphoreType.DMA,
    ],
    compiler_params=pltpu.CompilerParams(use_tc_tiling_on_sc=False),
)(_sc_edge_body)


def _tc_post_body(acca_ref, accb_ref, g_ref, b_ref, out_ref):
    aa = acca_ref[...]
    u = aa[:, :D]
    s = aa[:, D:D + 1]
    cnt = aa[:, D + 1:D + 2]
    p = accb_ref[:, :D]
    o = u / (s + 1e-16) + p / jnp.maximum(cnt, 1.0)
    mu = jnp.mean(o, axis=1, keepdims=True)
    var = jnp.mean((o - mu) ** 2, axis=1, keepdims=True)
    out_ref[...] = (o - mu) * lax.rsqrt(var + 1e-5) * g_ref[...] + b_ref[...]


_tc_post = pl.pallas_call(
    _tc_post_body,
    out_shape=jax.ShapeDtypeStruct((NPAD, D), jnp.float32),
)


def kernel(x, edge_index, W_proj, Wq, bq, Wk, bk, Wv, bv, gamma, beta):
    xs = x[0]
    src = edge_index[0].astype(jnp.int32)
    dst = edge_index[1].astype(jnp.int32)
    ta, tb = _tc_pre(xs, W_proj.T, Wq.T, bq[None, :], Wk.T, bk[None, :],
                     Wv.T, bv[None, :])
    acca, accb = _sc_edge(ta, tb, src, dst)
    out = _tc_post(acca, accb, gamma[None, :], beta[None, :])
    return out[:N][None]


# speedup vs baseline: 18.1948x; 1.8058x over previous
import functools
import math

import jax
import jax.numpy as jnp
from jax import lax
from jax.experimental import pallas as pl
from jax.experimental.pallas import tpu as pltpu
from jax.experimental.pallas import tpu_sc as plsc

N = 10000
E = 320000
D = 128
W = 144
NPAD = 10240
NSUB = 16
EDGES_PER_SUB = E // NSUB
CHUNK = 80
NCHUNK = EDGES_PER_SUB // CHUNK
NBUF = 2
BC = 50
NBLK = NCHUNK // BC
ROWS_PER_SUB = NPAD // NSUB
INV_SQRT_HD = 1.0 / math.sqrt(16.0)


def _tc_pre_body(x_ref, wp_ref, wq_ref, bq_ref, wk_ref, bk_ref, wv_ref,
                 bv_ref, ta_ref, tb_ref):
    xv = x_ref[...]
    h = jnp.dot(xv, wp_ref[...], preferred_element_type=jnp.float32)
    q = jnp.dot(h, wq_ref[...], preferred_element_type=jnp.float32) + bq_ref[...]
    k = jnp.dot(h, wk_ref[...], preferred_element_type=jnp.float32) + bk_ref[...]
    v = jnp.dot(h, wv_ref[...], preferred_element_type=jnp.float32) + bv_ref[...]
    a = jnp.sum(q * k, axis=1, keepdims=True) * INV_SQRT_HD
    wa = jnp.exp(a - jnp.max(a))
    ones = jnp.ones((N, 1), jnp.float32)
    zpad14 = jnp.zeros((N, 14), jnp.float32)
    zpad16 = jnp.zeros((N, 16), jnp.float32)
    ta_ref[...] = jnp.concatenate([v * wa, wa, ones, zpad14], axis=1)
    tb_ref[...] = jnp.concatenate([h, zpad16], axis=1)


_tc_pre = pl.pallas_call(
    _tc_pre_body,
    out_shape=[
        jax.ShapeDtypeStruct((N, W), jnp.float32),
        jax.ShapeDtypeStruct((N, W), jnp.float32),
    ],
)


def _sc_edge_body(ta_hbm, tb_hbm, src_hbm, dst_hbm, outa_hbm, outb_hbm,
                  acc, src_v, dst_v, rows_v, zbuf, sem):
    cid = lax.axis_index("c")
    sid = lax.axis_index("s")

    for r in range(16):
        for c in range(W // 16):
            zbuf[r, pl.ds(c * 16, 16)] = jnp.zeros((16,), jnp.float32)
    for j in range(ROWS_PER_SUB // 16):
        pltpu.sync_copy(zbuf, acc.at[pl.ds(sid * ROWS_PER_SUB + j * 16, 16)])

    plsc.subcore_barrier()

    def block(blk, carry):
        idx_base = pl.ds(sid * NCHUNK + blk * BC, BC)
        pltpu.sync_copy(src_hbm.at[idx_base], src_v)
        pltpu.sync_copy(dst_hbm.at[idx_base], dst_v)

        def run(tbl):
            def outer(o, c):
                handles = [
                    pltpu.async_copy(tbl.at[src_v.at[o * NBUF + b]],
                                     rows_v.at[b], sem)
                    for b in range(NBUF)
                ]
                for b, hd in enumerate(handles):
                    hd.wait()
                    pltpu.sync_copy(rows_v.at[b],
                                    acc.at[dst_v.at[o * NBUF + b]], add=True)
                return c

            lax.fori_loop(0, BC // NBUF, outer, 0)

        @pl.when(cid == 0)
        def _():
            run(ta_hbm)

        @pl.when(cid == 1)
        def _():
            run(tb_hbm)

        return carry

    lax.fori_loop(0, NBLK, block, 0)
    plsc.subcore_barrier()

    dst_slice = pl.ds(sid * ROWS_PER_SUB, ROWS_PER_SUB)

    @pl.when(cid == 0)
    def _():
        pltpu.sync_copy(acc.at[dst_slice], outa_hbm.at[dst_slice])

    @pl.when(cid == 1)
    def _():
        pltpu.sync_copy(acc.at[dst_slice], outb_hbm.at[dst_slice])


_sc_edge = functools.partial(
    pl.kernel,
    out_type=[
        jax.ShapeDtypeStruct((NPAD, W), jnp.float32),
        jax.ShapeDtypeStruct((NPAD, W), jnp.float32),
    ],
    mesh=plsc.VectorSubcoreMesh(core_axis_name="c", subcore_axis_name="s"),
    scratch_types=[
        pltpu.VMEM_SHARED((NPAD, W), jnp.float32),
        pltpu.VMEM((BC, CHUNK), jnp.int32),
        pltpu.VMEM((BC, CHUNK), jnp.int32),
        pltpu.VMEM((NBUF, CHUNK, W), jnp.float32),
        pltpu.VMEM((16, W), jnp.float32),
        pltpu.SemaphoreType.DMA,
    ],
    compiler_params=pltpu.CompilerParams(use_tc_tiling_on_sc=False),
)(_sc_edge_body)


def _tc_post_body(acca_ref, accb_ref, g_ref, b_ref, out_ref):
    aa = acca_ref[...]
    u = aa[:, :D]
    s = aa[:, D:D + 1]
    cnt = aa[:, D + 1:D + 2]
    p = accb_ref[:, :D]
    o = u / (s + 1e-16) + p / jnp.maximum(cnt, 1.0)
    mu = jnp.mean(o, axis=1, keepdims=True)
    var = jnp.mean((o - mu) ** 2, axis=1, keepdims=True)
    out_ref[...] = (o - mu) * lax.rsqrt(var + 1e-5) * g_ref[...] + b_ref[...]


_tc_post = pl.pallas_call(
    _tc_post_body,
    out_shape=jax.ShapeDtypeStruct((NPAD, D), jnp.float32),
)


def kernel(x, edge_index, W_proj, Wq, bq, Wk, bk, Wv, bv, gamma, beta):
    xs = x[0]
    src = edge_index[0].astype(jnp.int32).reshape(E // CHUNK, CHUNK)
    dst = edge_index[1].astype(jnp.int32).reshape(E // CHUNK, CHUNK)
    ta, tb = _tc_pre(xs, W_proj.T, Wq.T, bq[None, :], Wk.T, bk[None, :],
                     Wv.T, bv[None, :])
    acca, accb = _sc_edge(ta, tb, src, dst)
    out = _tc_post(acca, accb, gamma[None, :], beta[None, :])
    return out[:N][None]


# speedup vs baseline: 22.4892x; 1.2360x over previous
import functools
import math

import jax
import jax.numpy as jnp
from jax import lax
from jax.experimental import pallas as pl
from jax.experimental.pallas import tpu as pltpu
from jax.experimental.pallas import tpu_sc as plsc

N = 10000
E = 320000
D = 128
W = 144
NPAD = 10240
NSUB = 16
EDGES_PER_SUB = E // NSUB
CHUNK = 80
NCHUNK = EDGES_PER_SUB // CHUNK
NBUF = 3
BC = 25
NBLK = NCHUNK // BC
ROWS_PER_SUB = NPAD // NSUB
INV_SQRT_HD = 1.0 / math.sqrt(16.0)


def _tc_pre_body(x_ref, wp_ref, wq_ref, bq_ref, wk_ref, bk_ref, wv_ref,
                 bv_ref, ta_ref, tb_ref):
    xv = x_ref[...]
    h = jnp.dot(xv, wp_ref[...], preferred_element_type=jnp.float32)
    q = jnp.dot(h, wq_ref[...], preferred_element_type=jnp.float32) + bq_ref[...]
    k = jnp.dot(h, wk_ref[...], preferred_element_type=jnp.float32) + bk_ref[...]
    v = jnp.dot(h, wv_ref[...], preferred_element_type=jnp.float32) + bv_ref[...]
    a = jnp.sum(q * k, axis=1, keepdims=True) * INV_SQRT_HD
    wa = jnp.exp(a - jnp.max(a))
    ones = jnp.ones((N, 1), jnp.float32)
    zpad14 = jnp.zeros((N, 14), jnp.float32)
    zpad16 = jnp.zeros((N, 16), jnp.float32)
    ta_ref[...] = jnp.concatenate([v * wa, wa, ones, zpad14], axis=1)
    tb_ref[...] = jnp.concatenate([h, zpad16], axis=1)


_tc_pre = pl.pallas_call(
    _tc_pre_body,
    out_shape=[
        jax.ShapeDtypeStruct((N, W), jnp.float32),
        jax.ShapeDtypeStruct((N, W), jnp.float32),
    ],
)


def _sc_edge_body(ta_hbm, tb_hbm, src_hbm, dst_hbm, outa_hbm, outb_hbm,
                  acc, src_v, dst_v, rows_v, sem_g, sem_s):
    cid = lax.axis_index("c")
    sid = lax.axis_index("s")

    def zrow(r, c):
        for col in range(W // 16):
            rows_v[0, r, pl.ds(col * 16, 16)] = jnp.zeros((16,), jnp.float32)
        return c

    lax.fori_loop(0, CHUNK, zrow, 0)
    for j in range(ROWS_PER_SUB // CHUNK):
        pltpu.sync_copy(
            rows_v.at[0], acc.at[pl.ds(sid * ROWS_PER_SUB + j * CHUNK, CHUNK)])

    plsc.subcore_barrier()

    def block(blk, carry):
        idx_base = pl.ds(sid * NCHUNK + blk * BC, BC)
        pltpu.sync_copy(src_hbm.at[idx_base], src_v)
        pltpu.sync_copy(dst_hbm.at[idx_base], dst_v)

        def run(tbl):
            for b in range(2):
                pltpu.async_copy(tbl.at[src_v.at[b]], rows_v.at[b], sem_g)

            def it(j, c):
                pltpu.make_async_copy(tbl.at[src_v.at[j]],
                                      rows_v.at[j % NBUF], sem_g).wait()

                @pl.when(j >= 1)
                def _():
                    pltpu.make_async_copy(
                        rows_v.at[(j - 1) % NBUF],
                        acc.at[dst_v.at[j - 1]], sem_s).wait()

                pltpu.async_copy(rows_v.at[j % NBUF],
                                 acc.at[dst_v.at[j]], sem_s, add=True)

                @pl.when(j + 2 < BC)
                def _():
                    pltpu.async_copy(tbl.at[src_v.at[j + 2]],
                                     rows_v.at[(j + 2) % NBUF], sem_g)

                return c

            lax.fori_loop(0, BC, it, 0)
            pltpu.make_async_copy(rows_v.at[(BC - 1) % NBUF],
                                  acc.at[dst_v.at[BC - 1]], sem_s).wait()

        @pl.when(cid == 0)
        def _():
            run(ta_hbm)

        @pl.when(cid == 1)
        def _():
            run(tb_hbm)

        return carry

    lax.fori_loop(0, NBLK, block, 0)
    plsc.subcore_barrier()

    dst_slice = pl.ds(sid * ROWS_PER_SUB, ROWS_PER_SUB)

    @pl.when(cid == 0)
    def _():
        pltpu.sync_copy(acc.at[dst_slice], outa_hbm.at[dst_slice])

    @pl.when(cid == 1)
    def _():
        pltpu.sync_copy(acc.at[dst_slice], outb_hbm.at[dst_slice])


_sc_edge = functools.partial(
    pl.kernel,
    out_type=[
        jax.ShapeDtypeStruct((NPAD, W), jnp.float32),
        jax.ShapeDtypeStruct((NPAD, W), jnp.float32),
    ],
    mesh=plsc.VectorSubcoreMesh(core_axis_name="c", subcore_axis_name="s"),
    scratch_types=[
        pltpu.VMEM_SHARED((NPAD, W), jnp.float32),
        pltpu.VMEM((BC, CHUNK), jnp.int32),
        pltpu.VMEM((BC, CHUNK), jnp.int32),
        pltpu.VMEM((NBUF, CHUNK, W), jnp.float32),
        pltpu.SemaphoreType.DMA,
        pltpu.SemaphoreType.DMA,
    ],
    compiler_params=pltpu.CompilerParams(use_tc_tiling_on_sc=False),
)(_sc_edge_body)


def _tc_post_body(acca_ref, accb_ref, g_ref, b_ref, out_ref):
    aa = acca_ref[...]
    u = aa[:, :D]
    s = aa[:, D:D + 1]
    cnt = aa[:, D + 1:D + 2]
    p = accb_ref[:, :D]
    o = u / (s + 1e-16) + p / jnp.maximum(cnt, 1.0)
    mu = jnp.mean(o, axis=1, keepdims=True)
    var = jnp.mean((o - mu) ** 2, axis=1, keepdims=True)
    out_ref[...] = (o - mu) * lax.rsqrt(var + 1e-5) * g_ref[...] + b_ref[...]


_tc_post = pl.pallas_call(
    _tc_post_body,
    out_shape=jax.ShapeDtypeStruct((NPAD, D), jnp.float32),
)


def kernel(x, edge_index, W_proj, Wq, bq, Wk, bk, Wv, bv, gamma, beta):
    xs = x[0]
    src = edge_index[0].astype(jnp.int32).reshape(E // CHUNK, CHUNK)
    dst = edge_index[1].astype(jnp.int32).reshape(E // CHUNK, CHUNK)
    ta, tb = _tc_pre(xs, W_proj.T, Wq.T, bq[None, :], Wk.T, bk[None, :],
                     Wv.T, bv[None, :])
    acca, accb = _sc_edge(ta, tb, src, dst)
    out = _tc_post(acca, accb, gamma[None, :], beta[None, :])
    return out[:N][None]


# speedup vs baseline: 23.6124x; 1.0499x over previous
import functools
import math

import jax
import jax.numpy as jnp
from jax import lax
from jax.experimental import pallas as pl
from jax.experimental.pallas import tpu as pltpu
from jax.experimental.pallas import tpu_sc as plsc

N = 10000
E = 320000
D = 128
W = 144
NPAD = 10240
NSUB = 16
EDGES_PER_SUB = E // NSUB
CHUNK = 40
NCHUNK = EDGES_PER_SUB // CHUNK
NBUF = 6
LOOKAHEAD = 5
BC = 50
NBLK = NCHUNK // BC
ROWS_PER_SUB = NPAD // NSUB
INV_SQRT_HD = 1.0 / math.sqrt(16.0)


def _tc_pre_body(x_ref, wp_ref, wq_ref, bq_ref, wk_ref, bk_ref, wv_ref,
                 bv_ref, ta_ref, tb_ref):
    xv = x_ref[...]
    h = jnp.dot(xv, wp_ref[...], preferred_element_type=jnp.float32)
    q = jnp.dot(h, wq_ref[...], preferred_element_type=jnp.float32) + bq_ref[...]
    k = jnp.dot(h, wk_ref[...], preferred_element_type=jnp.float32) + bk_ref[...]
    v = jnp.dot(h, wv_ref[...], preferred_element_type=jnp.float32) + bv_ref[...]
    a = jnp.sum(q * k, axis=1, keepdims=True) * INV_SQRT_HD
    wa = jnp.exp(a - jnp.max(a))
    ones = jnp.ones((N, 1), jnp.float32)
    zpad14 = jnp.zeros((N, 14), jnp.float32)
    zpad16 = jnp.zeros((N, 16), jnp.float32)
    ta_ref[...] = jnp.concatenate([v * wa, wa, ones, zpad14], axis=1)
    tb_ref[...] = jnp.concatenate([h, zpad16], axis=1)


_tc_pre = pl.pallas_call(
    _tc_pre_body,
    out_shape=[
        jax.ShapeDtypeStruct((N, W), jnp.float32),
        jax.ShapeDtypeStruct((N, W), jnp.float32),
    ],
)


def _sc_edge_body(ta_hbm, tb_hbm, src_hbm, dst_hbm, outa_hbm, outb_hbm,
                  acc, src_v, dst_v, rows_v, sem_g, sem_s):
    cid = lax.axis_index("c")
    sid = lax.axis_index("s")

    def zrow(r, c):
        for col in range(W // 16):
            rows_v[0, r, pl.ds(col * 16, 16)] = jnp.zeros((16,), jnp.float32)
        return c

    lax.fori_loop(0, CHUNK, zrow, 0)
    for j in range(ROWS_PER_SUB // CHUNK):
        pltpu.sync_copy(
            rows_v.at[0], acc.at[pl.ds(sid * ROWS_PER_SUB + j * CHUNK, CHUNK)])

    plsc.subcore_barrier()

    def block(blk, carry):
        idx_base = pl.ds(sid * NCHUNK + blk * BC, BC)
        pltpu.sync_copy(src_hbm.at[idx_base], src_v)
        pltpu.sync_copy(dst_hbm.at[idx_base], dst_v)

        def run(tbl):
            for b in range(LOOKAHEAD):
                pltpu.async_copy(tbl.at[src_v.at[b]], rows_v.at[b], sem_g)

            def it(j, c):
                pltpu.make_async_copy(tbl.at[src_v.at[j]],
                                      rows_v.at[j % NBUF], sem_g).wait()

                @pl.when(j >= 1)
                def _():
                    pltpu.make_async_copy(
                        rows_v.at[(j - 1) % NBUF],
                        acc.at[dst_v.at[j - 1]], sem_s).wait()

                pltpu.async_copy(rows_v.at[j % NBUF],
                                 acc.at[dst_v.at[j]], sem_s, add=True)

                @pl.when(j + LOOKAHEAD < BC)
                def _():
                    pltpu.async_copy(tbl.at[src_v.at[j + LOOKAHEAD]],
                                     rows_v.at[(j + LOOKAHEAD) % NBUF], sem_g)

                return c

            lax.fori_loop(0, BC, it, 0)
            pltpu.make_async_copy(rows_v.at[(BC - 1) % NBUF],
                                  acc.at[dst_v.at[BC - 1]], sem_s).wait()

        @pl.when(cid == 0)
        def _():
            run(ta_hbm)

        @pl.when(cid == 1)
        def _():
            run(tb_hbm)

        return carry

    lax.fori_loop(0, NBLK, block, 0)
    plsc.subcore_barrier()

    dst_slice = pl.ds(sid * ROWS_PER_SUB, ROWS_PER_SUB)

    @pl.when(cid == 0)
    def _():
        pltpu.sync_copy(acc.at[dst_slice], outa_hbm.at[dst_slice])

    @pl.when(cid == 1)
    def _():
        pltpu.sync_copy(acc.at[dst_slice], outb_hbm.at[dst_slice])


_sc_edge = functools.partial(
    pl.kernel,
    out_type=[
        jax.ShapeDtypeStruct((NPAD, W), jnp.float32),
        jax.ShapeDtypeStruct((NPAD, W), jnp.float32),
    ],
    mesh=plsc.VectorSubcoreMesh(core_axis_name="c", subcore_axis_name="s"),
    scratch_types=[
        pltpu.VMEM_SHARED((NPAD, W), jnp.float32),
        pltpu.VMEM((BC, CHUNK), jnp.int32),
        pltpu.VMEM((BC, CHUNK), jnp.int32),
        pltpu.VMEM((NBUF, CHUNK, W), jnp.float32),
        pltpu.SemaphoreType.DMA,
        pltpu.SemaphoreType.DMA,
    ],
    compiler_params=pltpu.CompilerParams(use_tc_tiling_on_sc=False),
)(_sc_edge_body)


def _tc_post_body(acca_ref, accb_ref, g_ref, b_ref, out_ref):
    aa = acca_ref[...]
    u = aa[:, :D]
    s = aa[:, D:D + 1]
    cnt = aa[:, D + 1:D + 2]
    p = accb_ref[:, :D]
    o = u / (s + 1e-16) + p / jnp.maximum(cnt, 1.0)
    mu = jnp.mean(o, axis=1, keepdims=True)
    var = jnp.mean((o - mu) ** 2, axis=1, keepdims=True)
    out_ref[...] = (o - mu) * lax.rsqrt(var + 1e-5) * g_ref[...] + b_ref[...]


_tc_post = pl.pallas_call(
    _tc_post_body,
    out_shape=jax.ShapeDtypeStruct((NPAD, D), jnp.float32),
)


def kernel(x, edge_index, W_proj, Wq, bq, Wk, bk, Wv, bv, gamma, beta):
    xs = x[0]
    src = edge_index[0].astype(jnp.int32).reshape(E // CHUNK, CHUNK)
    dst = edge_index[1].astype(jnp.int32).reshape(E // CHUNK, CHUNK)
    ta, tb = _tc_pre(xs, W_proj.T, Wq.T, bq[None, :], Wk.T, bk[None, :],
                     Wv.T, bv[None, :])
    acca, accb = _sc_edge(ta, tb, src, dst)
    out = _tc_post(acca, accb, gamma[None, :], beta[None, :])
    return out[:N][None]


# speedup vs baseline: 25.9731x; 1.1000x over previous
import functools
import math

import jax
import jax.numpy as jnp
from jax import lax
from jax.experimental import pallas as pl
from jax.experimental.pallas import tpu as pltpu
from jax.experimental.pallas import tpu_sc as plsc

N = 10000
E = 320000
D = 128
W = 288
NPAD = 10240
NSUB = 16
NWORKER = 32
EDGES_PER_SUB = E // NWORKER
CHUNK = 40
NCHUNK = EDGES_PER_SUB // CHUNK
NBUF = 6
LOOKAHEAD = 5
BC = 50
NBLK = NCHUNK // BC
ROWS_PER_SUB = NPAD // NSUB
INV_SQRT_HD = 1.0 / math.sqrt(16.0)


def _tc_pre_body(x_ref, wp_ref, wq_ref, bq_ref, wk_ref, bk_ref, wv_ref,
                 bv_ref, t_ref):
    xv = x_ref[...]
    h = jnp.dot(xv, wp_ref[...], preferred_element_type=jnp.float32)
    q = jnp.dot(h, wq_ref[...], preferred_element_type=jnp.float32) + bq_ref[...]
    k = jnp.dot(h, wk_ref[...], preferred_element_type=jnp.float32) + bk_ref[...]
    v = jnp.dot(h, wv_ref[...], preferred_element_type=jnp.float32) + bv_ref[...]
    a = jnp.sum(q * k, axis=1, keepdims=True) * INV_SQRT_HD
    wa = jnp.exp(a - jnp.max(a))
    ones = jnp.ones((N, 1), jnp.float32)
    zpad = jnp.zeros((N, W - 2 * D - 2), jnp.float32)
    t = jnp.concatenate([v * wa, h, wa, ones, zpad], axis=1)
    t_ref[...] = t.astype(jnp.bfloat16)


_tc_pre = pl.pallas_call(
    _tc_pre_body,
    out_shape=jax.ShapeDtypeStruct((N, W), jnp.bfloat16),
)


def _sc_edge_body(t_hbm, src_hbm, dst_hbm, outa_hbm, outb_hbm,
                  acc, src_v, dst_v, rows_v, sem_g, sem_s):
    cid = lax.axis_index("c")
    sid = lax.axis_index("s")
    wid = cid * NSUB + sid

    def zrow(r, c):
        for col in range(W // 32):
            rows_v[0, r, pl.ds(col * 32, 32)] = jnp.zeros((32,), jnp.bfloat16)
        return c

    lax.fori_loop(0, CHUNK, zrow, 0)
    for j in range(ROWS_PER_SUB // CHUNK):
        pltpu.sync_copy(
            rows_v.at[0], acc.at[pl.ds(sid * ROWS_PER_SUB + j * CHUNK, CHUNK)])

    plsc.subcore_barrier()

    def block(blk, carry):
        idx_base = pl.ds(wid * NCHUNK + blk * BC, BC)
        pltpu.sync_copy(src_hbm.at[idx_base], src_v)
        pltpu.sync_copy(dst_hbm.at[idx_base], dst_v)

        for b in range(LOOKAHEAD):
            pltpu.async_copy(t_hbm.at[src_v.at[b]], rows_v.at[b], sem_g)

        def it(j, c):
            pltpu.make_async_copy(t_hbm.at[src_v.at[j]],
                                  rows_v.at[j % NBUF], sem_g).wait()

            @pl.when(j >= 1)
            def _():
                pltpu.make_async_copy(
                    rows_v.at[(j - 1) % NBUF],
                    acc.at[dst_v.at[j - 1]], sem_s).wait()

            pltpu.async_copy(rows_v.at[j % NBUF],
                             acc.at[dst_v.at[j]], sem_s, add=True)

            @pl.when(j + LOOKAHEAD < BC)
            def _():
                pltpu.async_copy(t_hbm.at[src_v.at[j + LOOKAHEAD]],
                                 rows_v.at[(j + LOOKAHEAD) % NBUF], sem_g)

            return c

        lax.fori_loop(0, BC, it, 0)
        pltpu.make_async_copy(rows_v.at[(BC - 1) % NBUF],
                              acc.at[dst_v.at[BC - 1]], sem_s).wait()
        return carry

    lax.fori_loop(0, NBLK, block, 0)
    plsc.subcore_barrier()

    dst_slice = pl.ds(sid * ROWS_PER_SUB, ROWS_PER_SUB)

    @pl.when(cid == 0)
    def _():
        pltpu.sync_copy(acc.at[dst_slice], outa_hbm.at[dst_slice])

    @pl.when(cid == 1)
    def _():
        pltpu.sync_copy(acc.at[dst_slice], outb_hbm.at[dst_slice])


_sc_edge = functools.partial(
    pl.kernel,
    out_type=[
        jax.ShapeDtypeStruct((NPAD, W), jnp.bfloat16),
        jax.ShapeDtypeStruct((NPAD, W), jnp.bfloat16),
    ],
    mesh=plsc.VectorSubcoreMesh(core_axis_name="c", subcore_axis_name="s"),
    scratch_types=[
        pltpu.VMEM_SHARED((NPAD, W), jnp.bfloat16),
        pltpu.VMEM((BC, CHUNK), jnp.int32),
        pltpu.VMEM((BC, CHUNK), jnp.int32),
        pltpu.VMEM((NBUF, CHUNK, W), jnp.bfloat16),
        pltpu.SemaphoreType.DMA,
        pltpu.SemaphoreType.DMA,
    ],
    compiler_params=pltpu.CompilerParams(use_tc_tiling_on_sc=False),
)(_sc_edge_body)


def _tc_post_body(acca_ref, accb_ref, g_ref, b_ref, out_ref):
    t = (acca_ref[...].astype(jnp.float32)
         + accb_ref[...].astype(jnp.float32))
    u = t[:, :D]
    p = t[:, D:2 * D]
    s = t[:, 2 * D:2 * D + 1]
    cnt = t[:, 2 * D + 1:2 * D + 2]
    o = u / (s + 1e-16) + p / jnp.maximum(cnt, 1.0)
    mu = jnp.mean(o, axis=1, keepdims=True)
    var = jnp.mean((o - mu) ** 2, axis=1, keepdims=True)
    out_ref[...] = (o - mu) * lax.rsqrt(var + 1e-5) * g_ref[...] + b_ref[...]


_tc_post = pl.pallas_call(
    _tc_post_body,
    out_shape=jax.ShapeDtypeStruct((NPAD, D), jnp.float32),
)


def kernel(x, edge_index, W_proj, Wq, bq, Wk, bk, Wv, bv, gamma, beta):
    xs = x[0]
    src = edge_index[0].astype(jnp.int32).reshape(E // CHUNK, CHUNK)
    dst = edge_index[1].astype(jnp.int32).reshape(E // CHUNK, CHUNK)
    t = _tc_pre(xs, W_proj.T, Wq.T, bq[None, :], Wk.T, bk[None, :],
                Wv.T, bv[None, :])
    acca, accb = _sc_edge(t, src, dst)
    out = _tc_post(acca, accb, gamma[None, :], beta[None, :])
    return out[:N][None]


# speedup vs baseline: 26.0720x; 1.0038x over previous
import functools
import math

import jax
import jax.numpy as jnp
from jax import lax
from jax.experimental import pallas as pl
from jax.experimental.pallas import tpu as pltpu
from jax.experimental.pallas import tpu_sc as plsc

N = 10000
E = 320000
D = 128
W = 288
NPAD = 10240
NSUB = 16
NWORKER = 32
EDGES_PER_SUB = E // NWORKER
CHUNK = 40
NCHUNK = EDGES_PER_SUB // CHUNK
NBUF = 6
LOOKAHEAD = 4
SINFLT = 2
BC = 50
NBLK = NCHUNK // BC
ROWS_PER_SUB = NPAD // NSUB
INV_SQRT_HD = 1.0 / math.sqrt(16.0)


def _tc_pre_body(x_ref, wp_ref, wq_ref, bq_ref, wk_ref, bk_ref, wv_ref,
                 bv_ref, t_ref):
    xv = x_ref[...]
    h = jnp.dot(xv, wp_ref[...], preferred_element_type=jnp.float32)
    q = jnp.dot(h, wq_ref[...], preferred_element_type=jnp.float32) + bq_ref[...]
    k = jnp.dot(h, wk_ref[...], preferred_element_type=jnp.float32) + bk_ref[...]
    v = jnp.dot(h, wv_ref[...], preferred_element_type=jnp.float32) + bv_ref[...]
    a = jnp.sum(q * k, axis=1, keepdims=True) * INV_SQRT_HD
    wa = jnp.exp(a - jnp.max(a))
    ones = jnp.ones((N, 1), jnp.float32)
    zpad = jnp.zeros((N, W - 2 * D - 2), jnp.float32)
    t = jnp.concatenate([v * wa, h, wa, ones, zpad], axis=1)
    t_ref[...] = t.astype(jnp.bfloat16)


_tc_pre = pl.pallas_call(
    _tc_pre_body,
    out_shape=jax.ShapeDtypeStruct((N, W), jnp.bfloat16),
)


def _sc_edge_body(t_hbm, src_hbm, dst_hbm, outa_hbm, outb_hbm,
                  acc, src_v, dst_v, rows_v, sem_g, sem_s):
    cid = lax.axis_index("c")
    sid = lax.axis_index("s")
    wid = cid * NSUB + sid

    def zrow(r, c):
        for col in range(W // 32):
            rows_v[0, r, pl.ds(col * 32, 32)] = jnp.zeros((32,), jnp.bfloat16)
        return c

    lax.fori_loop(0, CHUNK, zrow, 0)
    for j in range(ROWS_PER_SUB // CHUNK):
        pltpu.sync_copy(
            rows_v.at[0], acc.at[pl.ds(sid * ROWS_PER_SUB + j * CHUNK, CHUNK)])

    plsc.subcore_barrier()

    def block(blk, carry):
        idx_base = pl.ds(wid * NCHUNK + blk * BC, BC)
        pltpu.sync_copy(src_hbm.at[idx_base], src_v)
        pltpu.sync_copy(dst_hbm.at[idx_base], dst_v)

        for b in range(LOOKAHEAD):
            pltpu.async_copy(t_hbm.at[src_v.at[b]], rows_v.at[b], sem_g)

        def it(j, c):
            pltpu.make_async_copy(t_hbm.at[src_v.at[j]],
                                  rows_v.at[j % NBUF], sem_g).wait()

            @pl.when(j >= SINFLT)
            def _():
                pltpu.make_async_copy(
                    rows_v.at[(j - SINFLT) % NBUF],
                    acc.at[dst_v.at[j - SINFLT]], sem_s).wait()

            pltpu.async_copy(rows_v.at[j % NBUF],
                             acc.at[dst_v.at[j]], sem_s, add=True)

            @pl.when(j + LOOKAHEAD < BC)
            def _():
                pltpu.async_copy(t_hbm.at[src_v.at[j + LOOKAHEAD]],
                                 rows_v.at[(j + LOOKAHEAD) % NBUF], sem_g)

            return c

        lax.fori_loop(0, BC, it, 0)
        for k in range(BC - SINFLT, BC):
            pltpu.make_async_copy(rows_v.at[k % NBUF],
                                  acc.at[dst_v.at[k]], sem_s).wait()
        return carry

    lax.fori_loop(0, NBLK, block, 0)
    plsc.subcore_barrier()

    dst_slice = pl.ds(sid * ROWS_PER_SUB, ROWS_PER_SUB)

    @pl.when(cid == 0)
    def _():
        pltpu.sync_copy(acc.at[dst_slice], outa_hbm.at[dst_slice])

    @pl.when(cid == 1)
    def _():
        pltpu.sync_copy(acc.at[dst_slice], outb_hbm.at[dst_slice])


_sc_edge = functools.partial(
    pl.kernel,
    out_type=[
        jax.ShapeDtypeStruct((NPAD, W), jnp.bfloat16),
        jax.ShapeDtypeStruct((NPAD, W), jnp.bfloat16),
    ],
    mesh=plsc.VectorSubcoreMesh(core_axis_name="c", subcore_axis_name="s"),
    scratch_types=[
        pltpu.VMEM_SHARED((NPAD, W), jnp.bfloat16),
        pltpu.VMEM((BC, CHUNK), jnp.int32),
        pltpu.VMEM((BC, CHUNK), jnp.int32),
        pltpu.VMEM((NBUF, CHUNK, W), jnp.bfloat16),
        pltpu.SemaphoreType.DMA,
        pltpu.SemaphoreType.DMA,
    ],
    compiler_params=pltpu.CompilerParams(use_tc_tiling_on_sc=False),
)(_sc_edge_body)


def _tc_post_body(acca_ref, accb_ref, g_ref, b_ref, out_ref):
    t = (acca_ref[...].astype(jnp.float32)
         + accb_ref[...].astype(jnp.float32))
    u = t[:, :D]
    p = t[:, D:2 * D]
    s = t[:, 2 * D:2 * D + 1]
    cnt = t[:, 2 * D + 1:2 * D + 2]
    o = u / (s + 1e-16) + p / jnp.maximum(cnt, 1.0)
    mu = jnp.mean(o, axis=1, keepdims=True)
    var = jnp.mean((o - mu) ** 2, axis=1, keepdims=True)
    out_ref[...] = (o - mu) * lax.rsqrt(var + 1e-5) * g_ref[...] + b_ref[...]


_tc_post = pl.pallas_call(
    _tc_post_body,
    out_shape=jax.ShapeDtypeStruct((NPAD, D), jnp.float32),
)


def kernel(x, edge_index, W_proj, Wq, bq, Wk, bk, Wv, bv, gamma, beta):
    xs = x[0]
    src = edge_index[0].astype(jnp.int32).reshape(E // CHUNK, CHUNK)
    dst = edge_index[1].astype(jnp.int32).reshape(E // CHUNK, CHUNK)
    t = _tc_pre(xs, W_proj.T, Wq.T, bq[None, :], Wk.T, bk[None, :],
                Wv.T, bv[None, :])
    acca, accb = _sc_edge(t, src, dst)
    out = _tc_post(acca, accb, gamma[None, :], beta[None, :])
    return out[:N][None]


# speedup vs baseline: 26.4945x; 1.0162x over previous
import functools
import math

import jax
import jax.numpy as jnp
from jax import lax
from jax.experimental import pallas as pl
from jax.experimental.pallas import tpu as pltpu
from jax.experimental.pallas import tpu_sc as plsc

N = 10000
E = 320000
D = 128
W = 272
NPAD = 10240
NSUB = 16
NWORKER = 32
EDGES_PER_SUB = E // NWORKER
CHUNK = 80
NCHUNK = EDGES_PER_SUB // CHUNK
NBUF = 3
LOOKAHEAD = 2
SINFLT = 1
BC = 25
NBLK = NCHUNK // BC
ROWS_PER_SUB = NPAD // NSUB
INV_SQRT_HD = 1.0 / math.sqrt(16.0)


def _mmt(a, w):
    return lax.dot_general(a, w, (((1,), (1,)), ((), ())),
                           preferred_element_type=jnp.float32)


def _tc_pre_body(x_ref, wp_ref, wq_ref, bq_ref, wk_ref, bk_ref, wv_ref,
                 bv_ref, t_ref):
    xv = x_ref[...]
    h = _mmt(xv, wp_ref[...])
    q = _mmt(h, wq_ref[...]) + bq_ref[...]
    k = _mmt(h, wk_ref[...]) + bk_ref[...]
    v = _mmt(h, wv_ref[...]) + bv_ref[...]
    a = jnp.sum(q * k, axis=1, keepdims=True) * INV_SQRT_HD
    wa = jnp.exp(a - jnp.max(a))
    ones = jnp.ones((N, 1), jnp.float32)
    zpad = jnp.zeros((N, W - 2 * D - 2), jnp.float32)
    t = jnp.concatenate([v * wa, h, wa, ones, zpad], axis=1)
    t_ref[...] = t.astype(jnp.bfloat16)


_tc_pre = pl.pallas_call(
    _tc_pre_body,
    out_shape=jax.ShapeDtypeStruct((N, W), jnp.bfloat16),
)


def _sc_edge_body(t_hbm, src_hbm, dst_hbm, zeros_hbm, outa_hbm, outb_hbm,
                  acc, src_v, dst_v, rows_v, sem_g, sem_s):
    cid = lax.axis_index("c")
    sid = lax.axis_index("s")
    wid = cid * NSUB + sid

    pltpu.sync_copy(zeros_hbm, rows_v.at[0])
    for j in range(ROWS_PER_SUB // CHUNK):
        pltpu.sync_copy(
            rows_v.at[0], acc.at[pl.ds(sid * ROWS_PER_SUB + j * CHUNK, CHUNK)])

    plsc.subcore_barrier()

    def block(blk, carry):
        idx_base = pl.ds(wid * NCHUNK + blk * BC, BC)
        pltpu.sync_copy(src_hbm.at[idx_base], src_v)
        pltpu.sync_copy(dst_hbm.at[idx_base], dst_v)

        for b in range(LOOKAHEAD):
            pltpu.async_copy(t_hbm.at[src_v.at[b]], rows_v.at[b], sem_g)

        def it(j, c):
            pltpu.make_async_copy(t_hbm.at[src_v.at[j]],
                                  rows_v.at[j % NBUF], sem_g).wait()

            @pl.when(j >= SINFLT)
            def _():
                pltpu.make_async_copy(
                    rows_v.at[(j - SINFLT) % NBUF],
                    acc.at[dst_v.at[j - SINFLT]], sem_s).wait()

            pltpu.async_copy(rows_v.at[j % NBUF],
                             acc.at[dst_v.at[j]], sem_s, add=True)

            @pl.when(j + LOOKAHEAD < BC)
            def _():
                pltpu.async_copy(t_hbm.at[src_v.at[j + LOOKAHEAD]],
                                 rows_v.at[(j + LOOKAHEAD) % NBUF], sem_g)

            return c

        lax.fori_loop(0, BC, it, 0)
        for k in range(BC - SINFLT, BC):
            pltpu.make_async_copy(rows_v.at[k % NBUF],
                                  acc.at[dst_v.at[k]], sem_s).wait()
        return carry

    lax.fori_loop(0, NBLK, block, 0)
    plsc.subcore_barrier()

    dst_slice = pl.ds(sid * ROWS_PER_SUB, ROWS_PER_SUB)

    @pl.when(cid == 0)
    def _():
        pltpu.sync_copy(acc.at[dst_slice], outa_hbm.at[dst_slice])

    @pl.when(cid == 1)
    def _():
        pltpu.sync_copy(acc.at[dst_slice], outb_hbm.at[dst_slice])


_sc_edge = functools.partial(
    pl.kernel,
    out_type=[
        jax.ShapeDtypeStruct((NPAD, W), jnp.bfloat16),
        jax.ShapeDtypeStruct((NPAD, W), jnp.bfloat16),
    ],
    mesh=plsc.VectorSubcoreMesh(core_axis_name="c", subcore_axis_name="s"),
    scratch_types=[
        pltpu.VMEM_SHARED((NPAD, W), jnp.bfloat16),
        pltpu.VMEM((BC, CHUNK), jnp.int32),
        pltpu.VMEM((BC, CHUNK), jnp.int32),
        pltpu.VMEM((NBUF, CHUNK, W), jnp.bfloat16),
        pltpu.SemaphoreType.DMA,
        pltpu.SemaphoreType.DMA,
    ],
    compiler_params=pltpu.CompilerParams(use_tc_tiling_on_sc=False),
)(_sc_edge_body)


def _tc_post_body(acca_ref, accb_ref, g_ref, b_ref, out_ref):
    t = (acca_ref[...].astype(jnp.float32)
         + accb_ref[...].astype(jnp.float32))
    u = t[:, :D]
    p = t[:, D:2 * D]
    s = t[:, 2 * D:2 * D + 1]
    cnt = t[:, 2 * D + 1:2 * D + 2]
    o = u / (s + 1e-16) + p / jnp.maximum(cnt, 1.0)
    mu = jnp.mean(o, axis=1, keepdims=True)
    var = jnp.mean((o - mu) ** 2, axis=1, keepdims=True)
    out_ref[...] = (o - mu) * lax.rsqrt(var + 1e-5) * g_ref[...] + b_ref[...]


_tc_post = pl.pallas_call(
    _tc_post_body,
    out_shape=jax.ShapeDtypeStruct((NPAD, D), jnp.float32),
)


def kernel(x, edge_index, W_proj, Wq, bq, Wk, bk, Wv, bv, gamma, beta):
    xs = x[0]
    src = edge_index[0].astype(jnp.int32).reshape(E // CHUNK, CHUNK)
    dst = edge_index[1].astype(jnp.int32).reshape(E // CHUNK, CHUNK)
    t = _tc_pre(xs, W_proj, Wq, bq[None, :], Wk, bk[None, :],
                Wv, bv[None, :])
    zeros = jnp.zeros((CHUNK, W), jnp.bfloat16)
    acca, accb = _sc_edge(t, src, dst, zeros)
    out = _tc_post(acca, accb, gamma[None, :], beta[None, :])
    return out[:N][None]


# speedup vs baseline: 26.9124x; 1.0158x over previous
import functools
import math

import jax
import jax.numpy as jnp
from jax import lax
from jax.experimental import pallas as pl
from jax.experimental.pallas import tpu as pltpu
from jax.experimental.pallas import tpu_sc as plsc

N = 10000
E = 320000
D = 128
W = 272
NPAD = 10240
NSUB = 16
NWORKER = 32
EDGES_PER_SUB = E // NWORKER
CHUNK = 80
NCHUNK = EDGES_PER_SUB // CHUNK
NBUF = 3
LOOKAHEAD = 2
SINFLT = 1
BC = 25
NBLK = NCHUNK // BC
ROWS_PER_SUB = NPAD // NSUB
INV_SQRT_HD = 1.0 / math.sqrt(16.0)


def _mmt(a, w):
    return lax.dot_general(a, w, (((1,), (1,)), ((), ())),
                           preferred_element_type=jnp.float32)


def _tc_pre_body(x_ref, wp_ref, wq_ref, bq_ref, wk_ref, bk_ref, wv_ref,
                 bv_ref, t_ref):
    xv = x_ref[...]
    h = _mmt(xv, wp_ref[...])
    q = _mmt(h, wq_ref[...]) + bq_ref[...]
    k = _mmt(h, wk_ref[...]) + bk_ref[...]
    v = _mmt(h, wv_ref[...]) + bv_ref[...]
    a = jnp.sum(q * k, axis=1, keepdims=True) * INV_SQRT_HD
    wa = jnp.exp(a - jnp.max(a))
    ones = jnp.ones((N, 1), jnp.float32)
    zpad = jnp.zeros((N, W - 2 * D - 2), jnp.float32)
    t = jnp.concatenate([v * wa, h, wa, ones, zpad], axis=1)
    t_ref[...] = t.astype(jnp.bfloat16)


_tc_pre = pl.pallas_call(
    _tc_pre_body,
    out_shape=jax.ShapeDtypeStruct((N, W), jnp.bfloat16),
)


def _sc_edge_body(t_hbm, src_hbm, dst_hbm, zeros_hbm, outa_hbm, outb_hbm,
                  acc, src_v, dst_v, rows_v, sem_g, sem_s):
    cid = lax.axis_index("c")
    sid = lax.axis_index("s")
    wid = cid * NSUB + sid

    pltpu.sync_copy(zeros_hbm, rows_v.at[0])
    for j in range(ROWS_PER_SUB // CHUNK):
        pltpu.sync_copy(
            rows_v.at[0], acc.at[pl.ds(sid * ROWS_PER_SUB + j * CHUNK, CHUNK)])

    plsc.subcore_barrier()

    def block(blk, carry):
        idx_base = pl.ds(wid * NCHUNK + blk * BC, BC)
        pltpu.sync_copy(src_hbm.at[idx_base], src_v)
        pltpu.sync_copy(dst_hbm.at[idx_base], dst_v)

        for b in range(LOOKAHEAD):
            pltpu.async_copy(t_hbm.at[src_v.at[b]], rows_v.at[b], sem_g)

        def it(j, c):
            pltpu.make_async_copy(t_hbm.at[src_v.at[j]],
                                  rows_v.at[j % NBUF], sem_g).wait()

            @pl.when(j >= SINFLT)
            def _():
                pltpu.make_async_copy(
                    rows_v.at[(j - SINFLT) % NBUF],
                    acc.at[dst_v.at[j - SINFLT]], sem_s).wait()

            pltpu.async_copy(rows_v.at[j % NBUF],
                             acc.at[dst_v.at[j]], sem_s, add=True)

            @pl.when(j + LOOKAHEAD < BC)
            def _():
                pltpu.async_copy(t_hbm.at[src_v.at[j + LOOKAHEAD]],
                                 rows_v.at[(j + LOOKAHEAD) % NBUF], sem_g)

            return c

        lax.fori_loop(0, BC, it, 0)
        for k in range(BC - SINFLT, BC):
            pltpu.make_async_copy(rows_v.at[k % NBUF],
                                  acc.at[dst_v.at[k]], sem_s).wait()
        return carry

    lax.fori_loop(0, NBLK, block, 0)
    plsc.subcore_barrier()

    dst_slice = pl.ds(sid * ROWS_PER_SUB, ROWS_PER_SUB)

    @pl.when(cid == 0)
    def _():
        pltpu.sync_copy(acc.at[dst_slice], outa_hbm.at[dst_slice])

    @pl.when(cid == 1)
    def _():
        pltpu.sync_copy(acc.at[dst_slice], outb_hbm.at[dst_slice])


_sc_edge = functools.partial(
    pl.kernel,
    out_type=[
        jax.ShapeDtypeStruct((NPAD, W), jnp.bfloat16),
        jax.ShapeDtypeStruct((NPAD, W), jnp.bfloat16),
    ],
    mesh=plsc.VectorSubcoreMesh(core_axis_name="c", subcore_axis_name="s"),
    scratch_types=[
        pltpu.VMEM_SHARED((NPAD, W), jnp.bfloat16),
        pltpu.VMEM((BC, CHUNK), jnp.int32),
        pltpu.VMEM((BC, CHUNK), jnp.int32),
        pltpu.VMEM((NBUF, CHUNK, W), jnp.bfloat16),
        pltpu.SemaphoreType.DMA,
        pltpu.SemaphoreType.DMA,
    ],
    compiler_params=pltpu.CompilerParams(use_tc_tiling_on_sc=False),
)(_sc_edge_body)


def _tc_post_body(acca_ref, accb_ref, g_ref, b_ref, out_ref):
    t = (acca_ref[...].astype(jnp.float32)
         + accb_ref[...].astype(jnp.float32))
    u = t[:, :D]
    p = t[:, D:2 * D]
    s = t[:, 2 * D:2 * D + 1]
    cnt = t[:, 2 * D + 1:2 * D + 2]
    o = u / (s + 1e-16) + p / jnp.maximum(cnt, 1.0)
    mu = jnp.mean(o, axis=1, keepdims=True)
    var = jnp.mean((o - mu) ** 2, axis=1, keepdims=True)
    r = (o - mu) * lax.rsqrt(var + 1e-5) * g_ref[...] + b_ref[...]
    out_ref[...] = r[:N][None]


_tc_post = pl.pallas_call(
    _tc_post_body,
    out_shape=jax.ShapeDtypeStruct((1, N, D), jnp.float32),
)


def kernel(x, edge_index, W_proj, Wq, bq, Wk, bk, Wv, bv, gamma, beta):
    xs = x[0]
    src = edge_index[0].astype(jnp.int32).reshape(E // CHUNK, CHUNK)
    dst = edge_index[1].astype(jnp.int32).reshape(E // CHUNK, CHUNK)
    t = _tc_pre(xs, W_proj, Wq, bq[None, :], Wk, bk[None, :],
                Wv, bv[None, :])
    zeros = jnp.zeros((CHUNK, W), jnp.bfloat16)
    acca, accb = _sc_edge(t, src, dst, zeros)
    return _tc_post(acca, accb, gamma[None, :], beta[None, :])
